# Initial kernel scaffold; baseline (speedup 1.0000x reference)
import functools

import jax
import jax.numpy as jnp
from jax import lax
from jax.experimental import pallas as pl
from jax.experimental.pallas import tpu as pltpu
from jax.experimental.pallas import tpu_sc as plsc

N = 10000
E = 320000
D_IN = 128
F = 32
OUT = 128

NC = 2
NS = 16
NW = NC * NS
CHUNK = 128
N_PAD = 10112
ROWS_PER_TILE = N_PAD // NS
EDGES_PER_W = 10240
NCHUNK = EDGES_PER_W // CHUNK
E_PAD = EDGES_PER_W * NW

_mesh = plsc.VectorSubcoreMesh(
    core_axis_name="c", subcore_axis_name="s", num_cores=NC, num_subcores=NS)


@functools.partial(
    pl.kernel,
    out_type=jax.ShapeDtypeStruct((NC, N_PAD, 8), jnp.float32),
    mesh=_mesh,
    scratch_types=[
        pltpu.VMEM((NCHUNK, CHUNK), jnp.int32),
        pltpu.VMEM((CHUNK, 8), jnp.float32),
        pltpu.VMEM_SHARED((N_PAD, 8), jnp.float32),
    ],
)
def _deg_kernel(dst_hbm, zeros_hbm, ones_hbm, out_hbm, dst_v, ones_v, deg_sh):
    c = lax.axis_index("c")
    s = lax.axis_index("s")
    wid = s * NC + c
    row0 = s * ROWS_PER_TILE
    pltpu.sync_copy(zeros_hbm.at[pl.ds(row0, ROWS_PER_TILE)],
                    deg_sh.at[pl.ds(row0, ROWS_PER_TILE)])
    pltpu.sync_copy(dst_hbm.at[wid], dst_v)
    pltpu.sync_copy(ones_hbm, ones_v)
    plsc.subcore_barrier()

    def body(j, _):
        pltpu.sync_copy(ones_v, deg_sh.at[dst_v.at[j]], add=True)
        return ()

    lax.fori_loop(0, NCHUNK, body, ())
    plsc.subcore_barrier()
    pltpu.sync_copy(deg_sh.at[pl.ds(row0, ROWS_PER_TILE)],
                    out_hbm.at[c, pl.ds(row0, ROWS_PER_TILE)])


@functools.partial(
    pl.kernel,
    out_type=jax.ShapeDtypeStruct((NC, N_PAD, F), jnp.float32),
    mesh=_mesh,
    scratch_types=[
        pltpu.VMEM((NCHUNK + 2, CHUNK), jnp.int32),
        pltpu.VMEM((NCHUNK, CHUNK), jnp.int32),
        pltpu.VMEM((2, CHUNK, F), jnp.float32),
        pltpu.SemaphoreType.DMA,
        pltpu.SemaphoreType.DMA,
        pltpu.VMEM_SHARED((N_PAD, F), jnp.float32),
    ],
)
def _agg_kernel(src_hbm, dst_hbm, hp_hbm, zeros_hbm, out_hbm,
                src_v, dst_v, rows_v, sem0, sem1, acc_sh):
    c = lax.axis_index("c")
    s = lax.axis_index("s")
    wid = s * NC + c
    row0 = s * ROWS_PER_TILE
    sems = [sem0, sem1]

    pltpu.sync_copy(zeros_hbm.at[pl.ds(row0, ROWS_PER_TILE)],
                    acc_sh.at[pl.ds(row0, ROWS_PER_TILE)])
    pltpu.sync_copy(src_hbm.at[wid], src_v)
    pltpu.sync_copy(dst_hbm.at[wid], dst_v)
    plsc.subcore_barrier()

    for b in range(2):
        pltpu.async_copy(hp_hbm.at[src_v.at[b]], rows_v.at[b], sems[b])

    def body(jj, _):
        for b in range(2):
            jb = jj * 2 + b
            pltpu.make_async_copy(
                hp_hbm.at[src_v.at[jb]], rows_v.at[b], sems[b]).wait()
            pltpu.sync_copy(rows_v.at[b], acc_sh.at[dst_v.at[jb]], add=True)
            pltpu.async_copy(hp_hbm.at[src_v.at[jb + 2]], rows_v.at[b],
                             sems[b])
        return ()

    lax.fori_loop(0, NCHUNK // 2, body, ())
    for b in range(2):
        pltpu.make_async_copy(
            hp_hbm.at[src_v.at[NCHUNK + b]], rows_v.at[b], sems[b]).wait()
    plsc.subcore_barrier()
    pltpu.sync_copy(acc_sh.at[pl.ds(row0, ROWS_PER_TILE)],
                    out_hbm.at[c, pl.ds(row0, ROWS_PER_TILE)])


_BLK = 1264


def _tc1_body(x_ref, w_ref, degp_ref, hp_ref, dinv_ref):
    deg = degp_ref[0] + degp_ref[1] + 1.0
    dinv = lax.rsqrt(deg)
    h = jnp.dot(x_ref[...], w_ref[...], preferred_element_type=jnp.float32)
    hp_ref[...] = h * dinv
    dinv_ref[...] = dinv


def _tc_stage1(x_pad, W1, degp):
    return pl.pallas_call(
        _tc1_body,
        grid=(N_PAD // _BLK,),
        in_specs=[
            pl.BlockSpec((_BLK, D_IN), lambda i: (i, 0)),
            pl.BlockSpec((D_IN, F), lambda i: (0, 0)),
            pl.BlockSpec((NC, _BLK, 1), lambda i: (0, i, 0)),
        ],
        out_specs=[
            pl.BlockSpec((_BLK, F), lambda i: (i, 0)),
            pl.BlockSpec((_BLK, 1), lambda i: (i, 0)),
        ],
        out_shape=[
            jax.ShapeDtypeStruct((N_PAD, F), jnp.float32),
            jax.ShapeDtypeStruct((N_PAD, 1), jnp.float32),
        ],
    )(x_pad, W1, degp)


def _tc_combine_body(p_ref, hp_ref, dinv_ref, b_ref, g_ref, be_ref, w_ref,
                     out_ref):
    dinv = dinv_ref[...]
    agg = (p_ref[0] + p_ref[1] + hp_ref[...]) * dinv + b_ref[...]
    h = jnp.maximum(agg, 0.0)
    mu = jnp.mean(h, axis=1, keepdims=True)
    var = jnp.mean((h - mu) * (h - mu), axis=1, keepdims=True)
    hn = (h - mu) * lax.rsqrt(var + 1e-5) * g_ref[...] + be_ref[...]
    out_ref[...] = jnp.dot(
        hn, w_ref[...], preferred_element_type=jnp.float32) * dinv


def _tc_combine(p, hp, dinv2, b, g, be, Wn):
    return pl.pallas_call(
        _tc_combine_body,
        grid=(N_PAD // _BLK,),
        in_specs=[
            pl.BlockSpec((NC, _BLK, F), lambda i: (0, i, 0)),
            pl.BlockSpec((_BLK, F), lambda i: (i, 0)),
            pl.BlockSpec((_BLK, 1), lambda i: (i, 0)),
            pl.BlockSpec((1, F), lambda i: (0, 0)),
            pl.BlockSpec((1, F), lambda i: (0, 0)),
            pl.BlockSpec((1, F), lambda i: (0, 0)),
            pl.BlockSpec((F, F), lambda i: (0, 0)),
        ],
        out_specs=pl.BlockSpec((_BLK, F), lambda i: (i, 0)),
        out_shape=jax.ShapeDtypeStruct((N_PAD, F), jnp.float32),
    )(p, hp, dinv2, b, g, be, Wn)


def _tc_final_body(p_ref, hp_ref, dinv_ref, b3_ref, wp1_ref, bp1_ref,
                   wp2_ref, bp2_ref, emb_ref, lsm_ref):
    dinv = dinv_ref[...]
    emb = (p_ref[0] + p_ref[1] + hp_ref[...]) * dinv + b3_ref[...]
    emb_ref[...] = emb
    r = jnp.maximum(emb, 0.0)
    t = jnp.dot(r, wp1_ref[...], preferred_element_type=jnp.float32)
    t = t + bp1_ref[...]
    u = jnp.dot(t, wp2_ref[...], preferred_element_type=jnp.float32)
    u = u + bp2_ref[...]
    m = jnp.max(u, axis=1, keepdims=True)
    lse = jnp.log(jnp.sum(jnp.exp(u - m), axis=1, keepdims=True)) + m
    lsm_ref[...] = u - lse


def _tc_final(p, hp, dinv2, b3, Wp1, bp1, Wp2, bp2):
    return pl.pallas_call(
        _tc_final_body,
        grid=(N_PAD // _BLK,),
        in_specs=[
            pl.BlockSpec((NC, _BLK, F), lambda i: (0, i, 0)),
            pl.BlockSpec((_BLK, F), lambda i: (i, 0)),
            pl.BlockSpec((_BLK, 1), lambda i: (i, 0)),
            pl.BlockSpec((1, F), lambda i: (0, 0)),
            pl.BlockSpec((F, F), lambda i: (0, 0)),
            pl.BlockSpec((1, F), lambda i: (0, 0)),
            pl.BlockSpec((F, OUT), lambda i: (0, 0)),
            pl.BlockSpec((1, OUT), lambda i: (0, 0)),
        ],
        out_specs=[
            pl.BlockSpec((_BLK, F), lambda i: (i, 0)),
            pl.BlockSpec((_BLK, OUT), lambda i: (i, 0)),
        ],
        out_shape=[
            jax.ShapeDtypeStruct((N_PAD, F), jnp.float32),
            jax.ShapeDtypeStruct((N_PAD, OUT), jnp.float32),
        ],
    )(p, hp, dinv2, b3, Wp1, bp1, Wp2, bp2)


def kernel(x, edge_index, W1, b1, g1, be1, W2, b2, g2, be2, W3, b3,
           Wp1, bp1, Wp2, bp2):
    src = edge_index[0]
    dst = edge_index[1]
    src_p = jnp.pad(src, (0, E_PAD - E)).reshape(NW, NCHUNK, CHUNK)
    src_p = jnp.pad(src_p, ((0, 0), (0, 2), (0, 0)))
    dst_p = jnp.pad(dst, (0, E_PAD - E),
                    constant_values=N).reshape(NW, NCHUNK, CHUNK)

    zeros8 = jnp.zeros((N_PAD, 8), jnp.float32)
    ones8 = jnp.ones((CHUNK, 8), jnp.float32)
    zerosF = jnp.zeros((N_PAD, F), jnp.float32)
    x_pad = jnp.pad(x, ((0, N_PAD - N), (0, 0)))

    degp8 = _deg_kernel(dst_p, zeros8, ones8)
    degp = degp8[:, :, :1]

    hp1, dinv2 = _tc_stage1(x_pad, W1, degp)

    p1 = _agg_kernel(src_p, dst_p, hp1, zerosF)
    hp2 = _tc_combine(p1, hp1, dinv2, b1.reshape(1, F), g1.reshape(1, F),
                      be1.reshape(1, F), W2)

    p2 = _agg_kernel(src_p, dst_p, hp2, zerosF)
    hp3 = _tc_combine(p2, hp2, dinv2, b2.reshape(1, F), g2.reshape(1, F),
                      be2.reshape(1, F), W3)

    p3 = _agg_kernel(src_p, dst_p, hp3, zerosF)
    emb, lsm = _tc_final(p3, hp3, dinv2, b3.reshape(1, F), Wp1,
                         bp1.reshape(1, F), Wp2, bp2.reshape(1, OUT))
    return (emb[:N], lsm[:N])


# speedup vs baseline: 15.1857x; 15.1857x over previous
import functools

import jax
import jax.numpy as jnp
from jax import lax
from jax.experimental import pallas as pl
from jax.experimental.pallas import tpu as pltpu
from jax.experimental.pallas import tpu_sc as plsc

N = 10000
E = 320000
D_IN = 128
F = 32
OUT = 128

NC = 2
NS = 16
NW = NC * NS
CHUNK = 128
N_PAD = 10112
ROWS_PER_TILE = N_PAD // NS
EDGES_PER_W = 10240
NCHUNK = EDGES_PER_W // CHUNK
E_PAD = EDGES_PER_W * NW

def _deg_body(dst_hbm, zeros_hbm, ones_hbm, out_hbm, dst_v, ones_v, deg_sh):
    c = lax.axis_index("c")
    s = lax.axis_index("s")
    wid = s * NC + c
    row0 = s * ROWS_PER_TILE
    pltpu.sync_copy(zeros_hbm.at[pl.ds(row0, ROWS_PER_TILE)],
                    deg_sh.at[pl.ds(row0, ROWS_PER_TILE)])
    pltpu.sync_copy(dst_hbm.at[wid], dst_v)
    pltpu.sync_copy(ones_hbm, ones_v)
    plsc.subcore_barrier()

    def body(j, _):
        pltpu.sync_copy(ones_v, deg_sh.at[dst_v.at[j]], add=True)
        return ()

    lax.fori_loop(0, NCHUNK, body, ())
    plsc.subcore_barrier()
    pltpu.sync_copy(deg_sh.at[pl.ds(row0, ROWS_PER_TILE)],
                    out_hbm.at[c, pl.ds(row0, ROWS_PER_TILE)])


def _agg_body(src_hbm, dst_hbm, hp_hbm, zeros_hbm, out_hbm,
              src_v, dst_v, rows_v, sem0, sem1, acc_sh):
    c = lax.axis_index("c")
    s = lax.axis_index("s")
    wid = s * NC + c
    row0 = s * ROWS_PER_TILE
    sems = [sem0, sem1]

    pltpu.sync_copy(zeros_hbm.at[pl.ds(row0, ROWS_PER_TILE)],
                    acc_sh.at[pl.ds(row0, ROWS_PER_TILE)])
    pltpu.sync_copy(src_hbm.at[wid], src_v)
    pltpu.sync_copy(dst_hbm.at[wid], dst_v)
    plsc.subcore_barrier()

    for b in range(2):
        pltpu.async_copy(hp_hbm.at[src_v.at[b]], rows_v.at[b], sems[b])

    def body(jj, _):
        for b in range(2):
            jb = jj * 2 + b
            pltpu.make_async_copy(
                hp_hbm.at[src_v.at[jb]], rows_v.at[b], sems[b]).wait()
            pltpu.sync_copy(rows_v.at[b], acc_sh.at[dst_v.at[jb]], add=True)
            pltpu.async_copy(hp_hbm.at[src_v.at[jb + 2]], rows_v.at[b],
                             sems[b])
        return ()

    lax.fori_loop(0, NCHUNK // 2, body, ())
    for b in range(2):
        pltpu.make_async_copy(
            hp_hbm.at[src_v.at[NCHUNK + b]], rows_v.at[b], sems[b]).wait()
    plsc.subcore_barrier()
    pltpu.sync_copy(acc_sh.at[pl.ds(row0, ROWS_PER_TILE)],
                    out_hbm.at[c, pl.ds(row0, ROWS_PER_TILE)])


@functools.lru_cache(maxsize=None)
def _sc_kernels():
    mesh = plsc.VectorSubcoreMesh(
        core_axis_name="c", subcore_axis_name="s",
        num_cores=NC, num_subcores=NS)
    params = pltpu.CompilerParams(use_tc_tiling_on_sc=False)
    deg_k = pl.kernel(
        _deg_body,
        out_type=jax.ShapeDtypeStruct((NC, N_PAD, 8), jnp.float32),
        mesh=mesh,
        compiler_params=params,
        scratch_types=[
            pltpu.VMEM((NCHUNK, CHUNK), jnp.int32),
            pltpu.VMEM((CHUNK, 8), jnp.float32),
            pltpu.VMEM_SHARED((N_PAD, 8), jnp.float32),
        ],
    )
    agg_k = pl.kernel(
        _agg_body,
        out_type=jax.ShapeDtypeStruct((NC, N_PAD, F), jnp.float32),
        mesh=mesh,
        compiler_params=params,
        scratch_types=[
            pltpu.VMEM((NCHUNK + 2, CHUNK), jnp.int32),
            pltpu.VMEM((NCHUNK, CHUNK), jnp.int32),
            pltpu.VMEM((2, CHUNK, F), jnp.float32),
            pltpu.SemaphoreType.DMA,
            pltpu.SemaphoreType.DMA,
            pltpu.VMEM_SHARED((N_PAD, F), jnp.float32),
        ],
    )
    return deg_k, agg_k


_BLK = 1264


def _tc1_body(x_ref, w_ref, degp_ref, hp_ref, dinv_ref):
    deg = degp_ref[0] + degp_ref[1] + 1.0
    dinv = lax.rsqrt(deg)
    h = jnp.dot(x_ref[...], w_ref[...], preferred_element_type=jnp.float32)
    hp_ref[...] = h * dinv
    dinv_ref[...] = dinv


def _tc_stage1(x_pad, W1, degp):
    return pl.pallas_call(
        _tc1_body,
        grid=(N_PAD // _BLK,),
        in_specs=[
            pl.BlockSpec((_BLK, D_IN), lambda i: (i, 0)),
            pl.BlockSpec((D_IN, F), lambda i: (0, 0)),
            pl.BlockSpec((NC, _BLK, 1), lambda i: (0, i, 0)),
        ],
        out_specs=[
            pl.BlockSpec((_BLK, F), lambda i: (i, 0)),
            pl.BlockSpec((_BLK, 1), lambda i: (i, 0)),
        ],
        out_shape=[
            jax.ShapeDtypeStruct((N_PAD, F), jnp.float32),
            jax.ShapeDtypeStruct((N_PAD, 1), jnp.float32),
        ],
    )(x_pad, W1, degp)


def _tc_combine_body(p_ref, hp_ref, dinv_ref, b_ref, g_ref, be_ref, w_ref,
                     out_ref):
    dinv = dinv_ref[...]
    agg = (p_ref[0] + p_ref[1] + hp_ref[...]) * dinv + b_ref[...]
    h = jnp.maximum(agg, 0.0)
    mu = jnp.mean(h, axis=1, keepdims=True)
    var = jnp.mean((h - mu) * (h - mu), axis=1, keepdims=True)
    hn = (h - mu) * lax.rsqrt(var + 1e-5) * g_ref[...] + be_ref[...]
    out_ref[...] = jnp.dot(
        hn, w_ref[...], preferred_element_type=jnp.float32) * dinv


def _tc_combine(p, hp, dinv2, b, g, be, Wn):
    return pl.pallas_call(
        _tc_combine_body,
        grid=(N_PAD // _BLK,),
        in_specs=[
            pl.BlockSpec((NC, _BLK, F), lambda i: (0, i, 0)),
            pl.BlockSpec((_BLK, F), lambda i: (i, 0)),
            pl.BlockSpec((_BLK, 1), lambda i: (i, 0)),
            pl.BlockSpec((1, F), lambda i: (0, 0)),
            pl.BlockSpec((1, F), lambda i: (0, 0)),
            pl.BlockSpec((1, F), lambda i: (0, 0)),
            pl.BlockSpec((F, F), lambda i: (0, 0)),
        ],
        out_specs=pl.BlockSpec((_BLK, F), lambda i: (i, 0)),
        out_shape=jax.ShapeDtypeStruct((N_PAD, F), jnp.float32),
    )(p, hp, dinv2, b, g, be, Wn)


def _tc_final_body(p_ref, hp_ref, dinv_ref, b3_ref, wp1_ref, bp1_ref,
                   wp2_ref, bp2_ref, emb_ref, lsm_ref):
    dinv = dinv_ref[...]
    emb = (p_ref[0] + p_ref[1] + hp_ref[...]) * dinv + b3_ref[...]
    emb_ref[...] = emb
    r = jnp.maximum(emb, 0.0)
    t = jnp.dot(r, wp1_ref[...], preferred_element_type=jnp.float32)
    t = t + bp1_ref[...]
    u = jnp.dot(t, wp2_ref[...], preferred_element_type=jnp.float32)
    u = u + bp2_ref[...]
    m = jnp.max(u, axis=1, keepdims=True)
    lse = jnp.log(jnp.sum(jnp.exp(u - m), axis=1, keepdims=True)) + m
    lsm_ref[...] = u - lse


def _tc_final(p, hp, dinv2, b3, Wp1, bp1, Wp2, bp2):
    return pl.pallas_call(
        _tc_final_body,
        grid=(N_PAD // _BLK,),
        in_specs=[
            pl.BlockSpec((NC, _BLK, F), lambda i: (0, i, 0)),
            pl.BlockSpec((_BLK, F), lambda i: (i, 0)),
            pl.BlockSpec((_BLK, 1), lambda i: (i, 0)),
            pl.BlockSpec((1, F), lambda i: (0, 0)),
            pl.BlockSpec((F, F), lambda i: (0, 0)),
            pl.BlockSpec((1, F), lambda i: (0, 0)),
            pl.BlockSpec((F, OUT), lambda i: (0, 0)),
            pl.BlockSpec((1, OUT), lambda i: (0, 0)),
        ],
        out_specs=[
            pl.BlockSpec((_BLK, F), lambda i: (i, 0)),
            pl.BlockSpec((_BLK, OUT), lambda i: (i, 0)),
        ],
        out_shape=[
            jax.ShapeDtypeStruct((N_PAD, F), jnp.float32),
            jax.ShapeDtypeStruct((N_PAD, OUT), jnp.float32),
        ],
    )(p, hp, dinv2, b3, Wp1, bp1, Wp2, bp2)


def kernel(x, edge_index, W1, b1, g1, be1, W2, b2, g2, be2, W3, b3,
           Wp1, bp1, Wp2, bp2):
    src = edge_index[0]
    dst = edge_index[1]
    src_p = jnp.pad(src, (0, E_PAD - E)).reshape(NW, NCHUNK, CHUNK)
    src_p = jnp.pad(src_p, ((0, 0), (0, 2), (0, 0)))
    dst_p = jnp.pad(dst, (0, E_PAD - E),
                    constant_values=N).reshape(NW, NCHUNK, CHUNK)

    zeros8 = jnp.zeros((N_PAD, 8), jnp.float32)
    ones8 = jnp.ones((CHUNK, 8), jnp.float32)
    zerosF = jnp.zeros((N_PAD, F), jnp.float32)
    x_pad = jnp.pad(x, ((0, N_PAD - N), (0, 0)))

    _deg_kernel, _agg_kernel = _sc_kernels()
    degp8 = _deg_kernel(dst_p, zeros8, ones8)
    degp = degp8[:, :, :1]

    hp1, dinv2 = _tc_stage1(x_pad, W1, degp)

    p1 = _agg_kernel(src_p, dst_p, hp1, zerosF)
    hp2 = _tc_combine(p1, hp1, dinv2, b1.reshape(1, F), g1.reshape(1, F),
                      be1.reshape(1, F), W2)

    p2 = _agg_kernel(src_p, dst_p, hp2, zerosF)
    hp3 = _tc_combine(p2, hp2, dinv2, b2.reshape(1, F), g2.reshape(1, F),
                      be2.reshape(1, F), W3)

    p3 = _agg_kernel(src_p, dst_p, hp3, zerosF)
    emb, lsm = _tc_final(p3, hp3, dinv2, b3.reshape(1, F), Wp1,
                         bp1.reshape(1, F), Wp2, bp2.reshape(1, OUT))
    return (emb[:N], lsm[:N])
